# baseline (speedup 1.0000x reference)
import functools

import jax
import jax.numpy as jnp
from jax import lax
from jax.experimental import pallas as pl
from jax.experimental.pallas import tpu as pltpu
from jax.experimental.pallas import tpu_sc as plsc

N_NODES = 10000
D_FEAT = 128
N_EDGES = 320000
N_ACC = 10016

BLK = 128
NC = 2
NS = 16
NW = NC * NS
BPW = 80
NBLK = NW * BPW
E_PAD = NBLK * BLK
ROWS_MAIN = 632
ROWS_LAST = N_ACC - 15 * ROWS_MAIN


def _prep_body(x_ref, w_ref, ei_ref, emb_ref, idx_ref):
    z = x_ref[...] * w_ref[...]
    emb_ref[...] = jnp.where(z > 0, z, jnp.exp(z) - 1.0)
    i = pl.program_id(0)
    rblk = i * 256 + lax.broadcasted_iota(jnp.int32, (256, BLK), 0)
    gid = rblk * BLK + lax.broadcasted_iota(jnp.int32, (256, BLK), 1)
    real = rblk < N_EDGES // BLK
    idx_ref[:, 0, :] = jnp.where(real, ei_ref[0], gid % N_NODES)
    idx_ref[:, 1, :] = jnp.where(real, ei_ref[1],
                                 N_NODES + gid % (N_ACC - N_NODES))


def _prep(x, w, ei3):
    return pl.pallas_call(
        _prep_body,
        grid=(10,),
        in_specs=[
            pl.BlockSpec((1000, D_FEAT), lambda i: (i, 0)),
            pl.BlockSpec((1, D_FEAT), lambda i: (0, 0)),
            pl.BlockSpec((2, 256, BLK), lambda i: (0, i, 0)),
        ],
        out_specs=[
            pl.BlockSpec((1000, D_FEAT), lambda i: (i, 0)),
            pl.BlockSpec((256, 2, BLK), lambda i: (i, 0, 0)),
        ],
        out_shape=[
            jax.ShapeDtypeStruct((N_NODES, D_FEAT), jnp.float32),
            jax.ShapeDtypeStruct((NBLK, 2, BLK), jnp.int32),
        ],
    )(x, w, ei3)


def _combine_body(p_ref, o_ref):
    o_ref[...] = p_ref[0] + p_ref[1]


def _combine(p):
    return pl.pallas_call(
        _combine_body,
        grid=(10,),
        in_specs=[pl.BlockSpec((2, 1000, D_FEAT), lambda i: (0, i, 0))],
        out_specs=pl.BlockSpec((1000, D_FEAT), lambda i: (i, 0)),
        out_shape=jax.ShapeDtypeStruct((N_NODES, D_FEAT), jnp.float32),
    )(p)


def _sc_scatter(emb, idx_il, zrows):
    mesh = plsc.VectorSubcoreMesh(core_axis_name="c", subcore_axis_name="s")

    @functools.partial(
        pl.kernel,
        out_type=jax.ShapeDtypeStruct((NC, N_ACC, D_FEAT), jnp.float32),
        mesh=mesh,
        scratch_types=[
            pltpu.VMEM_SHARED((N_ACC, D_FEAT), jnp.float32),
            pltpu.VMEM((6, 2, BLK), jnp.int32),
            pltpu.VMEM((3, BLK, D_FEAT), jnp.float32),
            pltpu.SemaphoreType.DMA,
            pltpu.SemaphoreType.DMA,
            pltpu.SemaphoreType.DMA,
            pltpu.SemaphoreType.DMA,
            pltpu.SemaphoreType.DMA,
            pltpu.SemaphoreType.DMA,
            pltpu.SemaphoreType.DMA,
            pltpu.SemaphoreType.DMA,
            pltpu.SemaphoreType.DMA,
            pltpu.SemaphoreType.DMA,
            pltpu.SemaphoreType.DMA,
            pltpu.SemaphoreType.DMA,
        ],
    )
    def k(emb_hbm, idx_hbm, zrows_hbm, out_hbm, acc, idxr, rows,
          gsem0, gsem1, gsem2, ssem0, ssem1, ssem2,
          isem0, isem1, isem2, isem3, isem4, isem5):
        cid = lax.axis_index("c")
        sid = lax.axis_index("s")
        wid = sid * NC + cid
        wb = wid * BPW
        gsems = (gsem0, gsem1, gsem2)
        ssems = (ssem0, ssem1, ssem2)
        isems = (isem0, isem1, isem2, isem3, isem4, isem5)

        @pl.when(sid < NS - 1)
        def _():
            pltpu.sync_copy(zrows_hbm,
                            acc.at[pl.ds(sid * ROWS_MAIN, ROWS_MAIN)])

        @pl.when(sid == NS - 1)
        def _():
            pltpu.sync_copy(zrows_hbm.at[pl.ds(0, ROWS_LAST)],
                            acc.at[pl.ds(15 * ROWS_MAIN, ROWS_LAST)])

        plsc.subcore_barrier()

        def i_start(j, slot):
            pltpu.async_copy(idx_hbm.at[wb + j], idxr.at[slot], isems[slot])

        def i_wait(j, slot):
            pltpu.make_async_copy(idx_hbm.at[wb + j], idxr.at[slot],
                                  isems[slot]).wait()

        def g_start(j, d, slot, b):
            pltpu.async_copy(emb_hbm.at[idxr.at[slot, d]], rows.at[b],
                             gsems[b])

        def g_wait(j, d, slot, b):
            pltpu.make_async_copy(emb_hbm.at[idxr.at[slot, d]], rows.at[b],
                                  gsems[b]).wait()

        def s_start(j, d, slot, b):
            pltpu.async_copy(rows.at[b], acc.at[idxr.at[slot, 1 - d]],
                             ssems[b], add=True)

        def s_wait(j, d, slot, b):
            pltpu.make_async_copy(rows.at[b], acc.at[idxr.at[slot, 1 - d]],
                                  ssems[b]).wait()

        for j in range(4):
            i_start(j, j)
        i_wait(0, 0)
        g_start(0, 0, 0, 0)
        g_start(0, 1, 0, 1)
        g_wait(0, 0, 0, 0)
        s_start(0, 0, 0, 0)
        i_wait(1, 1)
        g_start(1, 0, 1, 2)
        g_wait(0, 1, 0, 1)
        s_start(0, 1, 0, 1)
        s_wait(0, 0, 0, 0)
        g_start(1, 1, 1, 0)
        g_wait(1, 0, 1, 2)
        s_start(1, 0, 1, 2)

        def body(m, _):
            jb = 2 + 6 * m
            for t in range(6):
                j = jb + t
                slot = (2 + t) % 6
                slot1 = (1 + t) % 6
                slot2 = (0 + t) % 6
                for d in (0, 1):
                    b = (4 + 2 * t + d) % 3
                    b1 = (3 + 2 * t + d) % 3
                    if d == 0:
                        s_wait(j - 2, 1, slot2, b)
                        i_wait(j, slot)
                        @pl.when(j < BPW - 2)
                        def _():
                            i_start(j + 2, (slot + 2) % 6)
                        g_start(j, 0, slot, b)
                        g_wait(j - 1, 1, slot1, b1)
                        s_start(j - 1, 1, slot1, b1)
                    else:
                        s_wait(j - 1, 0, slot1, b)
                        g_start(j, 1, slot, b)
                        g_wait(j, 0, slot, b1)
                        s_start(j, 0, slot, b1)
            return ()

        lax.fori_loop(0, 13, body, ())

        g_wait(79, 1, 1, 0)
        s_start(79, 1, 1, 0)
        s_wait(78, 1, 0, 1)
        s_wait(79, 0, 1, 2)
        s_wait(79, 1, 1, 0)
        plsc.subcore_barrier()

        @pl.when(sid < NS - 1)
        def _():
            pltpu.sync_copy(acc.at[pl.ds(sid * ROWS_MAIN, ROWS_MAIN)],
                            out_hbm.at[cid, pl.ds(sid * ROWS_MAIN, ROWS_MAIN)])

        @pl.when(sid == NS - 1)
        def _():
            pltpu.sync_copy(acc.at[pl.ds(15 * ROWS_MAIN, ROWS_LAST)],
                            out_hbm.at[cid, pl.ds(15 * ROWS_MAIN, ROWS_LAST)])

    return k(emb, idx_il, zrows)


def kernel(graph_embedding, edge_index, weight):
    ei3 = edge_index.astype(jnp.int32).reshape(2, N_EDGES // BLK, BLK)
    emb, idx_il = _prep(graph_embedding, weight, ei3)
    zrows = jnp.zeros((ROWS_MAIN, D_FEAT), jnp.float32)
    partial = _sc_scatter(emb, idx_il, zrows)
    return _combine(partial)


# speedup vs baseline: 1.0103x; 1.0103x over previous
import functools

import jax
import jax.numpy as jnp
from jax import lax
from jax.experimental import pallas as pl
from jax.experimental.pallas import tpu as pltpu
from jax.experimental.pallas import tpu_sc as plsc

N_NODES = 10000
D_FEAT = 128
N_EDGES = 320000
N_ACC = 10016

BLK = 128
NC = 2
NS = 16
NW = NC * NS
BPW = 80
NBLK = NW * BPW
E_PAD = NBLK * BLK
ROWS_MAIN = 632
ROWS_LAST = N_ACC - 15 * ROWS_MAIN


def _prep_body(x_ref, w_ref, ei_ref, emb_ref, idx_ref):
    z = x_ref[...] * w_ref[...]
    emb_ref[...] = jnp.where(z > 0, z, jnp.exp(z) - 1.0)
    i = pl.program_id(0)
    rblk = i * 256 + lax.broadcasted_iota(jnp.int32, (256, BLK), 0)
    gid = rblk * BLK + lax.broadcasted_iota(jnp.int32, (256, BLK), 1)
    real = rblk < N_EDGES // BLK
    idx_ref[:, 0, :] = jnp.where(real, ei_ref[0], gid % N_NODES)
    idx_ref[:, 1, :] = jnp.where(real, ei_ref[1],
                                 N_NODES + gid % (N_ACC - N_NODES))


def _prep(x, w, ei3):
    return pl.pallas_call(
        _prep_body,
        grid=(10,),
        in_specs=[
            pl.BlockSpec((1000, D_FEAT), lambda i: (i, 0)),
            pl.BlockSpec((1, D_FEAT), lambda i: (0, 0)),
            pl.BlockSpec((2, 256, BLK), lambda i: (0, i, 0)),
        ],
        out_specs=[
            pl.BlockSpec((1000, D_FEAT), lambda i: (i, 0)),
            pl.BlockSpec((256, 2, BLK), lambda i: (i, 0, 0)),
        ],
        out_shape=[
            jax.ShapeDtypeStruct((N_NODES, D_FEAT), jnp.float32),
            jax.ShapeDtypeStruct((NBLK, 2, BLK), jnp.int32),
        ],
    )(x, w, ei3)


def _combine_body(p_ref, o_ref):
    o_ref[...] = p_ref[0] + p_ref[1]


def _combine(p):
    return pl.pallas_call(
        _combine_body,
        grid=(10,),
        in_specs=[pl.BlockSpec((2, 1000, D_FEAT), lambda i: (0, i, 0))],
        out_specs=pl.BlockSpec((1000, D_FEAT), lambda i: (i, 0)),
        out_shape=jax.ShapeDtypeStruct((N_NODES, D_FEAT), jnp.float32),
    )(p)


def _sc_scatter(emb, idx_il, zrows):
    mesh = plsc.VectorSubcoreMesh(core_axis_name="c", subcore_axis_name="s")

    @functools.partial(
        pl.kernel,
        out_type=jax.ShapeDtypeStruct((NC, N_ACC, D_FEAT), jnp.float32),
        mesh=mesh,
        compiler_params=pltpu.CompilerParams(use_tc_tiling_on_sc=False),
        scratch_types=[
            pltpu.VMEM_SHARED((N_ACC, D_FEAT), jnp.float32),
            pltpu.VMEM((6, 2, BLK), jnp.int32),
            pltpu.VMEM((3, BLK, D_FEAT), jnp.float32),
            pltpu.SemaphoreType.DMA,
            pltpu.SemaphoreType.DMA,
            pltpu.SemaphoreType.DMA,
            pltpu.SemaphoreType.DMA,
            pltpu.SemaphoreType.DMA,
            pltpu.SemaphoreType.DMA,
            pltpu.SemaphoreType.DMA,
            pltpu.SemaphoreType.DMA,
            pltpu.SemaphoreType.DMA,
            pltpu.SemaphoreType.DMA,
            pltpu.SemaphoreType.DMA,
            pltpu.SemaphoreType.DMA,
        ],
    )
    def k(emb_hbm, idx_hbm, zrows_hbm, out_hbm, acc, idxr, rows,
          gsem0, gsem1, gsem2, ssem0, ssem1, ssem2,
          isem0, isem1, isem2, isem3, isem4, isem5):
        cid = lax.axis_index("c")
        sid = lax.axis_index("s")
        wid = sid * NC + cid
        wb = wid * BPW
        gsems = (gsem0, gsem1, gsem2)
        ssems = (ssem0, ssem1, ssem2)
        isems = (isem0, isem1, isem2, isem3, isem4, isem5)

        @pl.when(sid < NS - 1)
        def _():
            pltpu.sync_copy(zrows_hbm,
                            acc.at[pl.ds(sid * ROWS_MAIN, ROWS_MAIN)])

        @pl.when(sid == NS - 1)
        def _():
            pltpu.sync_copy(zrows_hbm.at[pl.ds(0, ROWS_LAST)],
                            acc.at[pl.ds(15 * ROWS_MAIN, ROWS_LAST)])

        plsc.subcore_barrier()

        def i_start(j, slot):
            pltpu.async_copy(idx_hbm.at[wb + j], idxr.at[slot], isems[slot])

        def i_wait(j, slot):
            pltpu.make_async_copy(idx_hbm.at[wb + j], idxr.at[slot],
                                  isems[slot]).wait()

        def g_start(j, d, slot, b):
            pltpu.async_copy(emb_hbm.at[idxr.at[slot, d]], rows.at[b],
                             gsems[b])

        def g_wait(j, d, slot, b):
            pltpu.make_async_copy(emb_hbm.at[idxr.at[slot, d]], rows.at[b],
                                  gsems[b]).wait()

        def s_start(j, d, slot, b):
            pltpu.async_copy(rows.at[b], acc.at[idxr.at[slot, 1 - d]],
                             ssems[b], add=True)

        def s_wait(j, d, slot, b):
            pltpu.make_async_copy(rows.at[b], acc.at[idxr.at[slot, 1 - d]],
                                  ssems[b]).wait()

        for j in range(4):
            i_start(j, j)
        i_wait(0, 0)
        g_start(0, 0, 0, 0)
        g_start(0, 1, 0, 1)
        g_wait(0, 0, 0, 0)
        s_start(0, 0, 0, 0)
        i_wait(1, 1)
        g_start(1, 0, 1, 2)
        g_wait(0, 1, 0, 1)
        s_start(0, 1, 0, 1)
        s_wait(0, 0, 0, 0)
        g_start(1, 1, 1, 0)
        g_wait(1, 0, 1, 2)
        s_start(1, 0, 1, 2)

        def body(m, _):
            jb = 2 + 6 * m
            for t in range(6):
                j = jb + t
                slot = (2 + t) % 6
                slot1 = (1 + t) % 6
                slot2 = (0 + t) % 6
                for d in (0, 1):
                    b = (4 + 2 * t + d) % 3
                    b1 = (3 + 2 * t + d) % 3
                    if d == 0:
                        s_wait(j - 2, 1, slot2, b)
                        i_wait(j, slot)
                        @pl.when(j < BPW - 2)
                        def _():
                            i_start(j + 2, (slot + 2) % 6)
                        g_start(j, 0, slot, b)
                        g_wait(j - 1, 1, slot1, b1)
                        s_start(j - 1, 1, slot1, b1)
                    else:
                        s_wait(j - 1, 0, slot1, b)
                        g_start(j, 1, slot, b)
                        g_wait(j, 0, slot, b1)
                        s_start(j, 0, slot, b1)
            return ()

        lax.fori_loop(0, 13, body, ())

        g_wait(79, 1, 1, 0)
        s_start(79, 1, 1, 0)
        s_wait(78, 1, 0, 1)
        s_wait(79, 0, 1, 2)
        s_wait(79, 1, 1, 0)
        plsc.subcore_barrier()

        @pl.when(sid < NS - 1)
        def _():
            pltpu.sync_copy(acc.at[pl.ds(sid * ROWS_MAIN, ROWS_MAIN)],
                            out_hbm.at[cid, pl.ds(sid * ROWS_MAIN, ROWS_MAIN)])

        @pl.when(sid == NS - 1)
        def _():
            pltpu.sync_copy(acc.at[pl.ds(15 * ROWS_MAIN, ROWS_LAST)],
                            out_hbm.at[cid, pl.ds(15 * ROWS_MAIN, ROWS_LAST)])

    return k(emb, idx_il, zrows)


def kernel(graph_embedding, edge_index, weight):
    ei3 = edge_index.astype(jnp.int32).reshape(2, N_EDGES // BLK, BLK)
    emb, idx_il = _prep(graph_embedding, weight, ei3)
    zrows = jnp.zeros((ROWS_MAIN, D_FEAT), jnp.float32)
    partial = _sc_scatter(emb, idx_il, zrows)
    return _combine(partial)


# speedup vs baseline: 1.0308x; 1.0203x over previous
import functools

import jax
import jax.numpy as jnp
from jax import lax
from jax.experimental import pallas as pl
from jax.experimental.pallas import tpu as pltpu
from jax.experimental.pallas import tpu_sc as plsc

N_NODES = 10000
D_FEAT = 128
N_EDGES = 320000
N_ACC = 10016

BLK = 128
NC = 2
NS = 16
NW = NC * NS
BPW = 80
NBLK = NW * BPW
E_PAD = NBLK * BLK
ROWS_MAIN = 632
ROWS_LAST = N_ACC - 15 * ROWS_MAIN


def _elu_body(x_ref, w_ref, o_ref):
    z = x_ref[...] * w_ref[...]
    o_ref[...] = jnp.where(z > 0, z, jnp.exp(z) - 1.0)


def _elu(x, w):
    return pl.pallas_call(
        _elu_body,
        grid=(10,),
        in_specs=[
            pl.BlockSpec((1000, D_FEAT), lambda i: (i, 0)),
            pl.BlockSpec((1, D_FEAT), lambda i: (0, 0)),
        ],
        out_specs=pl.BlockSpec((1000, D_FEAT), lambda i: (i, 0)),
        out_shape=jax.ShapeDtypeStruct((N_NODES, D_FEAT), jnp.float32),
    )(x, w)


def _combine_body(p_ref, o_ref):
    o_ref[...] = p_ref[0] + p_ref[1]


def _combine(p):
    return pl.pallas_call(
        _combine_body,
        grid=(10,),
        in_specs=[pl.BlockSpec((2, 1000, D_FEAT), lambda i: (0, i, 0))],
        out_specs=pl.BlockSpec((1000, D_FEAT), lambda i: (i, 0)),
        out_shape=jax.ShapeDtypeStruct((N_NODES, D_FEAT), jnp.float32),
    )(p)


def _sc_scatter(emb, idx_il, zrows):
    mesh = plsc.VectorSubcoreMesh(core_axis_name="c", subcore_axis_name="s")

    @functools.partial(
        pl.kernel,
        out_type=jax.ShapeDtypeStruct((NC, N_ACC, D_FEAT), jnp.float32),
        mesh=mesh,
        compiler_params=pltpu.CompilerParams(use_tc_tiling_on_sc=False),
        scratch_types=[
            pltpu.VMEM_SHARED((N_ACC, D_FEAT), jnp.float32),
            pltpu.VMEM((6, 2, BLK), jnp.int32),
            pltpu.VMEM((3, BLK, D_FEAT), jnp.float32),
            pltpu.SemaphoreType.DMA,
            pltpu.SemaphoreType.DMA,
            pltpu.SemaphoreType.DMA,
            pltpu.SemaphoreType.DMA,
            pltpu.SemaphoreType.DMA,
            pltpu.SemaphoreType.DMA,
            pltpu.SemaphoreType.DMA,
            pltpu.SemaphoreType.DMA,
            pltpu.SemaphoreType.DMA,
            pltpu.SemaphoreType.DMA,
            pltpu.SemaphoreType.DMA,
            pltpu.SemaphoreType.DMA,
        ],
    )
    def k(emb_hbm, idx_hbm, zrows_hbm, out_hbm, acc, idxr, rows,
          gsem0, gsem1, gsem2, ssem0, ssem1, ssem2,
          isem0, isem1, isem2, isem3, isem4, isem5):
        cid = lax.axis_index("c")
        sid = lax.axis_index("s")
        wid = sid * NC + cid
        wb = jnp.where(wid < 22, BPW * wid, 74 * wid + 132)
        T = jnp.where(wid < 22, 13, 12)
        gsems = (gsem0, gsem1, gsem2)
        ssems = (ssem0, ssem1, ssem2)
        isems = (isem0, isem1, isem2, isem3, isem4, isem5)

        @pl.when(sid < NS - 1)
        def _():
            pltpu.sync_copy(zrows_hbm,
                            acc.at[pl.ds(sid * ROWS_MAIN, ROWS_MAIN)])

        @pl.when(sid == NS - 1)
        def _():
            pltpu.sync_copy(zrows_hbm.at[pl.ds(0, ROWS_LAST)],
                            acc.at[pl.ds(15 * ROWS_MAIN, ROWS_LAST)])

        plsc.subcore_barrier()

        def i_start(j, slot):
            pltpu.async_copy(
                idx_hbm.at[pl.ds(0, 2), pl.ds(BLK * (wb + j), BLK)],
                idxr.at[slot], isems[slot])

        def i_wait(j, slot):
            pltpu.make_async_copy(
                idx_hbm.at[pl.ds(0, 2), pl.ds(BLK * (wb + j), BLK)],
                idxr.at[slot], isems[slot]).wait()

        def g_start(j, d, slot, b):
            pltpu.async_copy(emb_hbm.at[idxr.at[slot, d]], rows.at[b],
                             gsems[b])

        def g_wait(j, d, slot, b):
            pltpu.make_async_copy(emb_hbm.at[idxr.at[slot, d]], rows.at[b],
                                  gsems[b]).wait()

        def s_start(j, d, slot, b):
            pltpu.async_copy(rows.at[b], acc.at[idxr.at[slot, 1 - d]],
                             ssems[b], add=True)

        def s_wait(j, d, slot, b):
            pltpu.make_async_copy(rows.at[b], acc.at[idxr.at[slot, 1 - d]],
                                  ssems[b]).wait()

        for j in range(4):
            i_start(j, j)
        i_wait(0, 0)
        g_start(0, 0, 0, 0)
        g_start(0, 1, 0, 1)
        g_wait(0, 0, 0, 0)
        s_start(0, 0, 0, 0)
        i_wait(1, 1)
        g_start(1, 0, 1, 2)
        g_wait(0, 1, 0, 1)
        s_start(0, 1, 0, 1)
        s_wait(0, 0, 0, 0)
        g_start(1, 1, 1, 0)
        g_wait(1, 0, 1, 2)
        s_start(1, 0, 1, 2)

        def body(m, _):
            jb = 2 + 6 * m
            for t in range(6):
                j = jb + t
                slot = (2 + t) % 6
                slot1 = (1 + t) % 6
                slot2 = (0 + t) % 6
                for d in (0, 1):
                    b = (4 + 2 * t + d) % 3
                    b1 = (3 + 2 * t + d) % 3
                    if d == 0:
                        s_wait(j - 2, 1, slot2, b)
                        i_wait(j, slot)
                        @pl.when(j < 6 * T)
                        def _():
                            i_start(j + 2, (slot + 2) % 6)
                        g_start(j, 0, slot, b)
                        g_wait(j - 1, 1, slot1, b1)
                        s_start(j - 1, 1, slot1, b1)
                    else:
                        s_wait(j - 1, 0, slot1, b)
                        g_start(j, 1, slot, b)
                        g_wait(j, 0, slot, b1)
                        s_start(j, 0, slot, b1)
            return ()

        lax.fori_loop(0, T, body, ())

        jl = 6 * T + 1
        g_wait(jl, 1, 1, 0)
        s_start(jl, 1, 1, 0)
        s_wait(jl - 1, 1, 0, 1)
        s_wait(jl, 0, 1, 2)
        s_wait(jl, 1, 1, 0)
        plsc.subcore_barrier()

        @pl.when(sid < NS - 1)
        def _():
            pltpu.sync_copy(acc.at[pl.ds(sid * ROWS_MAIN, ROWS_MAIN)],
                            out_hbm.at[cid, pl.ds(sid * ROWS_MAIN, ROWS_MAIN)])

        @pl.when(sid == NS - 1)
        def _():
            pltpu.sync_copy(acc.at[pl.ds(15 * ROWS_MAIN, ROWS_LAST)],
                            out_hbm.at[cid, pl.ds(15 * ROWS_MAIN, ROWS_LAST)])

    return k(emb, idx_il, zrows)


def kernel(graph_embedding, edge_index, weight):
    emb = _elu(graph_embedding, weight)
    zrows = jnp.zeros((ROWS_MAIN, D_FEAT), jnp.float32)
    partial = _sc_scatter(emb, edge_index.astype(jnp.int32), zrows)
    return _combine(partial)


# speedup vs baseline: 1.0642x; 1.0324x over previous
import functools

import jax
import jax.numpy as jnp
from jax import lax
from jax.experimental import pallas as pl
from jax.experimental.pallas import tpu as pltpu
from jax.experimental.pallas import tpu_sc as plsc

N_NODES = 10000
D_FEAT = 128
N_EDGES = 320000
N_ACC = 10016

BLK = 128
NC = 2
NS = 16
NW = NC * NS
BPW = 80
NBLK = NW * BPW
E_PAD = NBLK * BLK
ROWS_MAIN = 632
ROWS_LAST = N_ACC - 15 * ROWS_MAIN


def _elu_body(x_ref, w_ref, o_ref):
    z = x_ref[...] * w_ref[...]
    o_ref[...] = jnp.where(z > 0, z, jnp.exp(z) - 1.0)


def _elu(x, w):
    return pl.pallas_call(
        _elu_body,
        grid=(10,),
        in_specs=[
            pl.BlockSpec((1000, D_FEAT), lambda i: (i, 0)),
            pl.BlockSpec((1, D_FEAT), lambda i: (0, 0)),
        ],
        out_specs=pl.BlockSpec((1000, D_FEAT), lambda i: (i, 0)),
        out_shape=jax.ShapeDtypeStruct((N_NODES, D_FEAT), jnp.float32),
    )(x, w)


def _combine_body(p_ref, o_ref):
    o_ref[...] = p_ref[0] + p_ref[1]


def _combine(p):
    return pl.pallas_call(
        _combine_body,
        grid=(10,),
        in_specs=[pl.BlockSpec((2, 1000, D_FEAT), lambda i: (0, i, 0))],
        out_specs=pl.BlockSpec((1000, D_FEAT), lambda i: (i, 0)),
        out_shape=jax.ShapeDtypeStruct((N_NODES, D_FEAT), jnp.float32),
    )(p)


def _sc_scatter(emb, idx_il):
    mesh = plsc.VectorSubcoreMesh(core_axis_name="c", subcore_axis_name="s")

    @functools.partial(
        pl.kernel,
        out_type=jax.ShapeDtypeStruct((NC, N_ACC, D_FEAT), jnp.float32),
        mesh=mesh,
        compiler_params=pltpu.CompilerParams(use_tc_tiling_on_sc=False),
        scratch_types=[
            pltpu.VMEM_SHARED((N_ACC, D_FEAT), jnp.float32),
            pltpu.VMEM((6, 2, BLK), jnp.int32),
            pltpu.VMEM((3, BLK, D_FEAT), jnp.float32),
            pltpu.SemaphoreType.DMA,
            pltpu.SemaphoreType.DMA,
            pltpu.SemaphoreType.DMA,
            pltpu.SemaphoreType.DMA,
            pltpu.SemaphoreType.DMA,
            pltpu.SemaphoreType.DMA,
            pltpu.SemaphoreType.DMA,
            pltpu.SemaphoreType.DMA,
            pltpu.SemaphoreType.DMA,
            pltpu.SemaphoreType.DMA,
            pltpu.SemaphoreType.DMA,
            pltpu.SemaphoreType.DMA,
        ],
    )
    def k(emb_hbm, idx_hbm, out_hbm, acc, idxr, rows,
          gsem0, gsem1, gsem2, ssem0, ssem1, ssem2,
          isem0, isem1, isem2, isem3, isem4, isem5):
        cid = lax.axis_index("c")
        sid = lax.axis_index("s")
        wid = sid * NC + cid
        wb = jnp.where(wid < 22, BPW * wid, 74 * wid + 132)
        T = jnp.where(wid < 22, 13, 12)
        gsems = (gsem0, gsem1, gsem2)
        ssems = (ssem0, ssem1, ssem2)
        isems = (isem0, isem1, isem2, isem3, isem4, isem5)

        def i_start(j, slot):
            pltpu.async_copy(
                idx_hbm.at[pl.ds(0, 2), pl.ds(BLK * (wb + j), BLK)],
                idxr.at[slot], isems[slot])

        def i_wait(j, slot):
            pltpu.make_async_copy(
                idx_hbm.at[pl.ds(0, 2), pl.ds(BLK * (wb + j), BLK)],
                idxr.at[slot], isems[slot]).wait()

        def g_start(j, d, slot, b):
            pltpu.async_copy(emb_hbm.at[idxr.at[slot, d]], rows.at[b],
                             gsems[b])

        def g_wait(j, d, slot, b):
            pltpu.make_async_copy(emb_hbm.at[idxr.at[slot, d]], rows.at[b],
                                  gsems[b]).wait()

        def s_start(j, d, slot, b):
            pltpu.async_copy(rows.at[b], acc.at[idxr.at[slot, 1 - d]],
                             ssems[b], add=True)

        def s_wait(j, d, slot, b):
            pltpu.make_async_copy(rows.at[b], acc.at[idxr.at[slot, 1 - d]],
                                  ssems[b]).wait()

        for j in range(4):
            i_start(j, j)
        i_wait(0, 0)
        g_start(0, 0, 0, 0)
        g_start(0, 1, 0, 1)

        def zfill(r, _):
            for c in range(8):
                rows[2, r, pl.ds(16 * c, 16)] = jnp.zeros((16,), jnp.float32)
            return ()

        lax.fori_loop(0, BLK, zfill, ())
        base = sid * ROWS_MAIN

        @pl.when(sid < NS - 1)
        def _():
            for off, sz in ((0, 128), (128, 128), (256, 128), (384, 128),
                            (512, ROWS_MAIN - 512)):
                pltpu.sync_copy(rows.at[2, pl.ds(0, sz)],
                                acc.at[pl.ds(base + off, sz)])

        @pl.when(sid == NS - 1)
        def _():
            for off, sz in ((0, 128), (128, 128), (256, 128), (384, 128),
                            (512, ROWS_LAST - 512)):
                pltpu.sync_copy(rows.at[2, pl.ds(0, sz)],
                                acc.at[pl.ds(base + off, sz)])

        plsc.subcore_barrier()
        g_wait(0, 0, 0, 0)
        s_start(0, 0, 0, 0)
        i_wait(1, 1)
        g_start(1, 0, 1, 2)
        g_wait(0, 1, 0, 1)
        s_start(0, 1, 0, 1)
        s_wait(0, 0, 0, 0)
        g_start(1, 1, 1, 0)
        g_wait(1, 0, 1, 2)
        s_start(1, 0, 1, 2)

        def body(m, _):
            jb = 2 + 6 * m
            for t in range(6):
                j = jb + t
                slot = (2 + t) % 6
                slot1 = (1 + t) % 6
                slot2 = (0 + t) % 6
                for d in (0, 1):
                    b = (4 + 2 * t + d) % 3
                    b1 = (3 + 2 * t + d) % 3
                    if d == 0:
                        s_wait(j - 2, 1, slot2, b)
                        i_wait(j, slot)
                        @pl.when(j < 6 * T)
                        def _():
                            i_start(j + 2, (slot + 2) % 6)
                        g_start(j, 0, slot, b)
                        g_wait(j - 1, 1, slot1, b1)
                        s_start(j - 1, 1, slot1, b1)
                    else:
                        s_wait(j - 1, 0, slot1, b)
                        g_start(j, 1, slot, b)
                        g_wait(j, 0, slot, b1)
                        s_start(j, 0, slot, b1)
            return ()

        lax.fori_loop(0, T, body, ())

        jl = 6 * T + 1
        g_wait(jl, 1, 1, 0)
        s_start(jl, 1, 1, 0)
        s_wait(jl - 1, 1, 0, 1)
        s_wait(jl, 0, 1, 2)
        s_wait(jl, 1, 1, 0)
        plsc.subcore_barrier()

        @pl.when(sid < NS - 1)
        def _():
            pltpu.sync_copy(acc.at[pl.ds(sid * ROWS_MAIN, ROWS_MAIN)],
                            out_hbm.at[cid, pl.ds(sid * ROWS_MAIN, ROWS_MAIN)])

        @pl.when(sid == NS - 1)
        def _():
            pltpu.sync_copy(acc.at[pl.ds(15 * ROWS_MAIN, ROWS_LAST)],
                            out_hbm.at[cid, pl.ds(15 * ROWS_MAIN, ROWS_LAST)])

    return k(emb, idx_il)


def kernel(graph_embedding, edge_index, weight):
    emb = _elu(graph_embedding, weight)
    partial = _sc_scatter(emb, edge_index.astype(jnp.int32))
    return _combine(partial)
